# baseline (speedup 1.0000x reference)
import functools

import jax
import jax.numpy as jnp
from jax import lax
from jax.experimental import pallas as pl
from jax.experimental.pallas import tpu as pltpu
from jax.experimental.pallas import tpu_sc as plsc

VOCAB = 30
DIM = 2048
NC = 2
NS = 16
NW = NC * NS
CHUNK = 16


@functools.lru_cache(maxsize=None)
def _make_gather(n_flat: int):
    b_per_w = n_flat // NW
    n_chunks = b_per_w // CHUNK
    assert n_flat == b_per_w * NW and b_per_w == n_chunks * CHUNK
    assert n_chunks % 2 == 0
    mesh = plsc.VectorSubcoreMesh(core_axis_name="c", subcore_axis_name="s")

    @functools.partial(
        pl.kernel,
        mesh=mesh,
        out_type=jax.ShapeDtypeStruct((n_flat, DIM), jnp.float32),
        scratch_types=[
            pltpu.VMEM((b_per_w,), jnp.int32),
            pltpu.VMEM((CHUNK, DIM), jnp.float32),
            pltpu.VMEM((CHUNK, DIM), jnp.float32),
            pltpu.SemaphoreType.DMA,
            pltpu.SemaphoreType.DMA,
            pltpu.SemaphoreType.DMA,
        ],
    )
    def k(idx_hbm, table_hbm, out_hbm, idx_v, buf0, buf1, gsem, ssem0, ssem1):
        wid = lax.axis_index("s") * NC + lax.axis_index("c")
        base = wid * b_per_w
        pltpu.sync_copy(idx_hbm.at[pl.ds(base, b_per_w)], idx_v)
        bufs = (buf0, buf1)
        ssems = (ssem0, ssem1)

        def step(g, carry):
            for b in range(2):
                c = 2 * g + b
                @pl.when(g > 0)
                def _wait_prev():
                    pltpu.make_async_copy(
                        bufs[b],
                        out_hbm.at[pl.ds(base + (c - 2) * CHUNK, CHUNK)],
                        ssems[b],
                    ).wait()

                pltpu.async_copy(
                    table_hbm.at[idx_v.at[pl.ds(c * CHUNK, CHUNK)]],
                    bufs[b],
                    gsem,
                ).wait()
                pltpu.make_async_copy(
                    bufs[b],
                    out_hbm.at[pl.ds(base + c * CHUNK, CHUNK)],
                    ssems[b],
                ).start()
            return carry

        lax.fori_loop(0, n_chunks // 2, step, 0)
        for b in range(2):
            c_last = n_chunks - 2 + b
            pltpu.make_async_copy(
                bufs[b],
                out_hbm.at[pl.ds(base + c_last * CHUNK, CHUNK)],
                ssems[b],
            ).wait()

    return k


def kernel(x, table):
    B, L = x.shape
    idx = x.reshape(B * L).astype(jnp.int32)
    out = _make_gather(B * L)(idx, table)
    return out.reshape(B, L, table.shape[1])


# speedup vs baseline: 1.6158x; 1.6158x over previous
import functools

import jax
import jax.numpy as jnp
from jax import lax
from jax.experimental import pallas as pl
from jax.experimental.pallas import tpu as pltpu
from jax.experimental.pallas import tpu_sc as plsc

VOCAB = 30
DIM = 2048
NC = 2
NS = 16
NW = NC * NS
LAG = 8


@functools.lru_cache(maxsize=None)
def _make_gather(n_flat: int):
    b_per_w = n_flat // NW
    assert n_flat == b_per_w * NW
    mesh = plsc.VectorSubcoreMesh(core_axis_name="c", subcore_axis_name="s")

    @functools.partial(
        pl.kernel,
        mesh=mesh,
        out_type=jax.ShapeDtypeStruct((n_flat, DIM), jnp.float32),
        scratch_types=[
            pltpu.VMEM((b_per_w,), jnp.int32),
            pltpu.VMEM((VOCAB, DIM), jnp.float32),
            pltpu.SemaphoreType.DMA,
        ],
    )
    def k(idx_hbm, table_hbm, out_hbm, idx_s, table_v, ssem):
        wid = lax.axis_index("s") * NC + lax.axis_index("c")
        base = wid * b_per_w
        pltpu.sync_copy(idx_hbm.at[pl.ds(base, b_per_w)], idx_s)
        pltpu.sync_copy(table_hbm, table_v)

        def issue(i, r):
            pltpu.make_async_copy(
                table_v.at[r], out_hbm.at[base + i], ssem
            ).start()

        def drain_one():
            pltpu.make_async_copy(
                table_v.at[0], out_hbm.at[base], ssem
            ).wait()

        def group(g, drain):
            v = idx_s[pl.ds(g * 16, 16)]
            for lane in range(16):
                if drain:
                    drain_one()
                issue(g * 16 + lane, v[lane])

        group(0, drain=False)

        def step(g, carry):
            group(g, drain=True)
            return carry

        lax.fori_loop(1, b_per_w // 16, step, 0)

        def drain_loop(i, carry):
            drain_one()
            return carry

        lax.fori_loop(0, 16, drain_loop, 0)

    return k


def kernel(x, table):
    B, L = x.shape
    idx = x.reshape(B * L).astype(jnp.int32)
    out = _make_gather(B * L)(idx, table)
    return out.reshape(B, L, table.shape[1])


# speedup vs baseline: 7.7567x; 4.8006x over previous
import functools

import jax
import jax.numpy as jnp
from jax import lax
from jax.experimental import pallas as pl
from jax.experimental.pallas import tpu as pltpu
from jax.experimental.pallas import tpu_sc as plsc

VOCAB = 30
DIM = 2048
NC = 2
NS = 16
NW = NC * NS
LAG = 8


@functools.lru_cache(maxsize=None)
def _make_gather(n_flat: int):
    b_per_w = n_flat // NW
    assert n_flat == b_per_w * NW
    mesh = plsc.VectorSubcoreMesh(core_axis_name="c", subcore_axis_name="s")

    @functools.partial(
        pl.kernel,
        mesh=mesh,
        out_type=jax.ShapeDtypeStruct((n_flat, DIM), jnp.float32),
        scratch_types=[
            pltpu.VMEM((b_per_w,), jnp.int32),
            pltpu.VMEM((VOCAB, DIM), jnp.float32),
            pltpu.SemaphoreType.DMA,
        ],
    )
    def k(idx_hbm, table_hbm, out_hbm, idx_s, table_v, ssem):
        wid = lax.axis_index("s") * NC + lax.axis_index("c")
        base = wid * b_per_w
        pltpu.sync_copy(idx_hbm.at[pl.ds(base, b_per_w)], idx_s)
        pltpu.sync_copy(table_hbm, table_v)

        def issue(i, r):
            pltpu.make_async_copy(
                table_v.at[r], out_hbm.at[base + i], ssem
            ).start()

        def drain_one():
            pltpu.make_async_copy(
                table_v.at[0], out_hbm.at[base], ssem
            ).wait()

        def group(g, drain):
            v = idx_s[pl.ds(g * 16, 16)]
            for lane in range(16):
                if drain:
                    drain_one()
                issue(g * 16 + lane, v[lane])

        group(0, drain=False)

        def step(g, carry):
            group(g, drain=True)
            return carry

        lax.fori_loop(1, b_per_w // 16, step, 0)

        def drain_loop(i, carry):
            drain_one()
            return carry

        lax.fori_loop(0, 16, drain_loop, 0)

    return k


def kernel(x, table):
    B, L = x.shape
    idx = x.T.reshape(B * L).astype(jnp.int32)
    out = _make_gather(B * L)(idx, table)
    return jnp.swapaxes(out.reshape(L, B, table.shape[1]), 0, 1)
